# Initial kernel scaffold; baseline (speedup 1.0000x reference)
import jax
import jax.numpy as jnp
from jax.experimental import pallas as pl


def kernel(drug_x, dis_x, drug_feat_src, drug_feat_dst, dis_feat_src, dis_feat_dst, d2d_src, d2d_dst, dr2ds_src, dr2ds_dst, W_drug_feat, a_drug_feat, W_dis_feat, a_dis_feat, W_bi_drug, W_bi_dis, a_bi_drug, a_bi_dis, gate_drug_W, gate_drug_b, gate_dis_W, gate_dis_b):
    raise NotImplementedError("write your pallas kernel here")


# speedup vs baseline: 2.7108x; 2.7108x over previous
import functools
import jax
import jax.numpy as jnp
from jax import lax
from jax.experimental import pallas as pl
from jax.experimental.pallas import tpu as pltpu
from jax.experimental.pallas import tpu_sc as plsc

N = 10000
D = 128
E = 320000
ND8 = 1280
NC = 2
NS = 16
NW = NC * NS
EW = E // NW
BE = 16
NBLK = EW // BE


def _sc_gat_body(rows_hbm, s1_hbm, s2_hbm, src_hbm, dst_hbm, zeros_hbm,
                 num_hbm, den_hbm, shnum, shden, s1_v, s2_v, srcb, dstb,
                 dst8b, expb, g128, sden, sem):
    c = lax.axis_index("c")
    s = lax.axis_index("s")
    w = s * NC + c

    @pl.when(s < NS - 1)
    def _():
        pltpu.sync_copy(zeros_hbm.at[pl.ds(s * 640, 640)],
                        shnum.at[pl.ds(s * 640, 640)])

    @pl.when(s == NS - 1)
    def _():
        pltpu.sync_copy(zeros_hbm.at[pl.ds(9600, 400)],
                        shnum.at[pl.ds(9600, 400)])

    pltpu.sync_copy(zeros_hbm.at[pl.ds(s * 80, 80)],
                    shden.at[pl.ds(s * 80, 80)])
    pltpu.sync_copy(zeros_hbm.at[pl.ds(0, BE)], sden)
    pltpu.sync_copy(s1_hbm, s1_v)
    pltpu.sync_copy(s2_hbm, s2_v)
    plsc.subcore_barrier()

    lane = lax.broadcasted_iota(jnp.int32, (16,), 0)

    def block_body(b, carry):
        base = w * EW + b * BE
        pltpu.sync_copy(src_hbm.at[pl.ds(base, BE)], srcb)
        pltpu.sync_copy(dst_hbm.at[pl.ds(base, BE)], dstb)
        pltpu.async_copy(rows_hbm.at[srcb], g128, sem).wait()
        for k in range(BE // 16):
            dsts = dstb[pl.ds(16 * k, 16)]
            srcs = srcb[pl.ds(16 * k, 16)]
            e = plsc.load_gather(s1_v, [dsts]) + plsc.load_gather(s2_v, [srcs])
            e = jnp.where(e > 0.0, e, 0.2 * e)
            ev = jnp.exp(e)
            expb[pl.ds(16 * k, 16)] = ev
            dst8b[pl.ds(16 * k, 16)] = dsts // 8
            plsc.store_scatter(sden, [lane + 16 * k, (dsts % 8) * 16 + lane],
                               ev)

        pltpu.sync_copy(sden, shden.at[dst8b], add=True)
        for k in range(BE // 16):
            dsts = dstb[pl.ds(16 * k, 16)]
            plsc.store_scatter(sden, [lane + 16 * k, (dsts % 8) * 16 + lane],
                               jnp.zeros((16,), jnp.float32))

        def row_body(r, carry2):
            rfull = jnp.full((16,), r, jnp.int32)
            ev = plsc.load_gather(expb, [rfull])
            for j in range(D // 16):
                col = lane + 16 * j
                v = plsc.load_gather(g128, [rfull, col])
                plsc.store_scatter(g128, [rfull, col], v * ev)
            return carry2

        lax.fori_loop(0, BE, row_body, 0)
        pltpu.sync_copy(g128, shnum.at[dstb], add=True)
        return carry

    lax.fori_loop(0, NBLK, block_body, 0)
    plsc.subcore_barrier()
    @pl.when(s < NS - 1)
    def _():
        pltpu.sync_copy(shnum.at[pl.ds(s * 640, 640)],
                        num_hbm.at[c, pl.ds(s * 640, 640)])

    @pl.when(s == NS - 1)
    def _():
        pltpu.sync_copy(shnum.at[pl.ds(9600, 400)],
                        num_hbm.at[c, pl.ds(9600, 400)])

    pltpu.sync_copy(shden.at[pl.ds(s * 80, 80)],
                    den_hbm.at[c, pl.ds(s * 80, 80)])


def _sc_gat(rows, s1, s2, src, dst, zeros):
    mesh = plsc.VectorSubcoreMesh(core_axis_name="c", subcore_axis_name="s")
    f = pl.kernel(
        _sc_gat_body, mesh=mesh,
        compiler_params=pltpu.CompilerParams(needs_layout_passes=False),
        out_type=[
            jax.ShapeDtypeStruct((NC, N, D), jnp.float32),
            jax.ShapeDtypeStruct((NC, ND8, D), jnp.float32),
        ],
        scratch_types=[
            pltpu.VMEM_SHARED((N, D), jnp.float32),
            pltpu.VMEM_SHARED((ND8, D), jnp.float32),
            pltpu.VMEM((N,), jnp.float32),
            pltpu.VMEM((N,), jnp.float32),
            pltpu.VMEM((BE,), jnp.int32),
            pltpu.VMEM((BE,), jnp.int32),
            pltpu.VMEM((BE,), jnp.int32),
            pltpu.VMEM((BE,), jnp.float32),
            pltpu.VMEM((BE, D), jnp.float32),
            pltpu.VMEM((BE, D), jnp.float32),
            pltpu.SemaphoreType.DMA,
        ],
    )
    return f(rows, s1, s2, src, dst, zeros)


def _proj_body(x_ref, w_ref, a_ref, h_ref, s_ref):
    h = jnp.dot(x_ref[...], w_ref[...], preferred_element_type=jnp.float32)
    h_ref[...] = h
    s_ref[...] = jnp.dot(h, a_ref[...], preferred_element_type=jnp.float32)


def _tc_proj(x, W, A2):
    blk = 2000
    return pl.pallas_call(
        _proj_body,
        grid=(N // blk,),
        in_specs=[
            pl.BlockSpec((blk, D), lambda i: (i, 0)),
            pl.BlockSpec((D, D), lambda i: (0, 0)),
            pl.BlockSpec((D, 2), lambda i: (0, 0)),
        ],
        out_specs=[
            pl.BlockSpec((blk, D), lambda i: (i, 0)),
            pl.BlockSpec((blk, 2), lambda i: (i, 0)),
        ],
        out_shape=[
            jax.ShapeDtypeStruct((N, D), jnp.float32),
            jax.ShapeDtypeStruct((N, 2), jnp.float32),
        ],
    )(x, W, A2)


def _den_col(dref):
    dsum = dref[0] + dref[1]
    dsum = dsum.reshape(ND8, 8, 16).sum(axis=2).reshape(ND8 * 8, 1)
    return dsum[:N]


def _fuse_body(fn_ref, fd_ref, bn_ref, bd_ref, gw_ref, gb_ref, o_ref):
    fden = _den_col(fd_ref)
    bden = _den_col(bd_ref)
    fnum = fn_ref[0] + fn_ref[1]
    bnum = bn_ref[0] + bn_ref[1]
    feat = jnp.maximum(fnum / (fden + 1e-16), 0.0)
    bi = jnp.maximum(bnum / (bden + 1e-16), 0.0)
    gw = gw_ref[...]
    g = jax.nn.sigmoid(
        jnp.dot(feat, gw[:, :D].T, preferred_element_type=jnp.float32)
        + jnp.dot(bi, gw[:, D:].T, preferred_element_type=jnp.float32)
        + gb_ref[0, 0])
    o_ref[...] = g * feat + (1.0 - g) * bi


def _tc_fuse(fnum, fden, bnum, bden, gate_W, gate_b):
    blk = N
    return pl.pallas_call(
        _fuse_body,
        grid=(N // blk,),
        in_specs=[
            pl.BlockSpec((NC, blk, D), lambda i: (0, i, 0)),
            pl.BlockSpec((NC, ND8, D), lambda i: (0, 0, 0)),
            pl.BlockSpec((NC, blk, D), lambda i: (0, i, 0)),
            pl.BlockSpec((NC, ND8, D), lambda i: (0, 0, 0)),
            pl.BlockSpec((1, 2 * D), lambda i: (0, 0)),
            pl.BlockSpec((1, 1), lambda i: (0, 0)),
        ],
        out_specs=pl.BlockSpec((blk, D), lambda i: (i, 0)),
        out_shape=jax.ShapeDtypeStruct((N, D), jnp.float32),
    )(fnum, fden, bnum, bden, gate_W, gate_b.reshape(1, 1))


def kernel(drug_x, dis_x, drug_feat_src, drug_feat_dst, dis_feat_src,
           dis_feat_dst, d2d_src, d2d_dst, dr2ds_src, dr2ds_dst,
           W_drug_feat, a_drug_feat, W_dis_feat, a_dis_feat,
           W_bi_drug, W_bi_dis, a_bi_drug, a_bi_dis,
           gate_drug_W, gate_drug_b, gate_dis_W, gate_dis_b):
    i32 = jnp.int32
    idx = [x.astype(i32) for x in
           (drug_feat_src, drug_feat_dst, dis_feat_src, dis_feat_dst,
            d2d_src, d2d_dst, dr2ds_src, dr2ds_dst)]
    (df_src, df_dst, pf_src, pf_dst, bd_src, bd_dst, bs_src, bs_dst) = idx

    A2_df = jnp.concatenate([a_drug_feat[:D], a_drug_feat[D:]], axis=1)
    A2_pf = jnp.concatenate([a_dis_feat[:D], a_dis_feat[D:]], axis=1)
    A2_bd = jnp.concatenate([a_bi_drug[:D], a_bi_dis[D:]], axis=1)
    A2_bs = jnp.concatenate([a_bi_dis[:D], a_bi_drug[D:]], axis=1)

    h_df, s_df = _tc_proj(drug_x, W_drug_feat, A2_df)
    h_pf, s_pf = _tc_proj(dis_x, W_dis_feat, A2_pf)
    h_bd, s_bd = _tc_proj(drug_x, W_bi_drug, A2_bd)
    h_bs, s_bs = _tc_proj(dis_x, W_bi_dis, A2_bs)

    z = jnp.zeros((N, D), jnp.float32)
    n_fd, d_fd = _sc_gat(h_df, s_df[:, 0], s_df[:, 1], df_src, df_dst, z)
    n_fp, d_fp = _sc_gat(h_pf, s_pf[:, 0], s_pf[:, 1], pf_src, pf_dst, z)
    n_bd, d_bd = _sc_gat(h_bs, s_bd[:, 0], s_bs[:, 1], bd_src, bd_dst, z)
    n_bp, d_bp = _sc_gat(h_bd, s_bs[:, 0], s_bd[:, 1], bs_src, bs_dst, z)

    hd = _tc_fuse(n_fd, d_fd, n_bd, d_bd, gate_drug_W, gate_drug_b)
    hp = _tc_fuse(n_fp, d_fp, n_bp, d_bp, gate_dis_W, gate_dis_b)
    return jnp.stack([hd, hp], axis=0)
